# baseline (speedup 1.0000x reference)
import functools

import jax
import jax.numpy as jnp
from jax import lax
from jax.experimental import pallas as pl
from jax.experimental.pallas import tpu as pltpu
from jax.experimental.pallas import tpu_sc as plsc

B = 4096 * 20
D = 512
NC, NS = 2, 16
NW = NC * NS
BPW = B // NW
CHUNK = 64
NCHUNK = BPW // CHUNK


def _proj_body(emb_ref, w_ref, b_ref, out_ref):
    out_ref[...] = lax.dot_general(
        emb_ref[...], w_ref[...],
        (((1,), (1,)), ((), ())),
        preferred_element_type=jnp.float32,
    ) + b_ref[...]


def _project(emb, W, b2):
    return pl.pallas_call(
        _proj_body,
        out_shape=jax.ShapeDtypeStruct((emb.shape[0], W.shape[0]), jnp.float32),
    )(emb, W, b2)


def _make_gather():
    mesh = plsc.VectorSubcoreMesh(core_axis_name="c", subcore_axis_name="s")

    @functools.partial(
        pl.kernel, mesh=mesh,
        out_type=jax.ShapeDtypeStruct((B, D), jnp.float32),
        scratch_types=[
            pltpu.VMEM((BPW,), jnp.int32),
            pltpu.VMEM((CHUNK, D), jnp.float32),
            pltpu.SemaphoreType.DMA,
        ],
    )
    def gather(table_hbm, idx_hbm, out_hbm, idx_v, rows_v, gsem):
        wid = lax.axis_index("s") * NC + lax.axis_index("c")
        base = wid * BPW
        pltpu.sync_copy(idx_hbm.at[pl.ds(base, BPW)], idx_v)

        def body(c, carry):
            pltpu.async_copy(
                table_hbm.at[idx_v.at[pl.ds(c * CHUNK, CHUNK)]],
                rows_v, gsem).wait()
            pltpu.sync_copy(rows_v, out_hbm.at[pl.ds(base + c * CHUNK, CHUNK)])
            return carry

        lax.fori_loop(0, NCHUNK, body, 0)

    return gather


_gather = _make_gather()


def kernel(x, emb_table, W, b):
    P = _project(emb_table, W, b.reshape(1, D))
    xf = x.reshape(-1).astype(jnp.int32)
    out = _gather(P, xf)
    return out.reshape(x.shape[0], x.shape[1], D)


# speedup vs baseline: 1.0055x; 1.0055x over previous
import functools

import jax
import jax.numpy as jnp
from jax import lax
from jax.experimental import pallas as pl
from jax.experimental.pallas import tpu as pltpu
from jax.experimental.pallas import tpu_sc as plsc

B = 4096 * 20
D = 512
NC, NS = 2, 16
NW = NC * NS
BPW = B // NW
CHUNK = 80
NCHUNK = BPW // CHUNK


def _proj_body(emb_ref, w_ref, b_ref, out_ref):
    out_ref[...] = lax.dot_general(
        emb_ref[...], w_ref[...],
        (((1,), (1,)), ((), ())),
        preferred_element_type=jnp.float32,
    ) + b_ref[...]


def _project(emb, W, b2):
    return pl.pallas_call(
        _proj_body,
        out_shape=jax.ShapeDtypeStruct((emb.shape[0], W.shape[0]), jnp.float32),
    )(emb, W, b2)


def _make_gather():
    mesh = plsc.VectorSubcoreMesh(core_axis_name="c", subcore_axis_name="s")

    @functools.partial(
        pl.kernel, mesh=mesh,
        out_type=jax.ShapeDtypeStruct((B, D), jnp.float32),
        scratch_types=[
            pltpu.VMEM((BPW,), jnp.int32),
            pltpu.VMEM((2, CHUNK, D), jnp.float32),
            pltpu.SemaphoreType.DMA,
            pltpu.SemaphoreType.DMA,
            pltpu.SemaphoreType.DMA,
        ],
    )
    def gather(table_hbm, idx_hbm, out_hbm, idx_v, rows_v, gsem, osem0, osem1):
        osem = (osem0, osem1)
        wid = lax.axis_index("s") * NC + lax.axis_index("c")
        base = wid * BPW
        pltpu.sync_copy(idx_hbm.at[pl.ds(base, BPW)], idx_v)

        def gdesc(c, b):
            return pltpu.make_async_copy(
                table_hbm.at[idx_v.at[pl.ds(c * CHUNK, CHUNK)]],
                rows_v.at[b], gsem)

        def odesc(c, b):
            return pltpu.make_async_copy(
                rows_v.at[b], out_hbm.at[pl.ds(base + c * CHUNK, CHUNK)],
                osem[b])

        gdesc(0, 0).start()

        def body(g, carry):
            for b in (0, 1):
                c = 2 * g + b
                gdesc(c, b).wait()

                @pl.when(c >= 1)
                def _():
                    odesc(c - 1, 1 - b).wait()

                odesc(c, b).start()

                @pl.when(c + 1 < NCHUNK)
                def _():
                    gdesc(c + 1, 1 - b).start()
            return carry

        lax.fori_loop(0, NCHUNK // 2, body, 0)
        odesc(NCHUNK - 1, (NCHUNK - 1) % 2).wait()

    return gather


_gather = _make_gather()


def kernel(x, emb_table, W, b):
    P = _project(emb_table, W, b.reshape(1, D))
    xf = x.reshape(-1).astype(jnp.int32)
    out = _gather(P, xf)
    return out.reshape(x.shape[0], x.shape[1], D)


# speedup vs baseline: 1.2171x; 1.2105x over previous
import functools

import jax
import jax.numpy as jnp
from jax import lax
from jax.experimental import pallas as pl
from jax.experimental.pallas import tpu as pltpu
from jax.experimental.pallas import tpu_sc as plsc

NB = 4096
L = 20
D = 512
NC, NS = 2, 16
NW = NC * NS
BPW = NB // NW
NBUF = 4
DEPTH = 3


def _proj_body(emb_ref, w_ref, b_ref, out_ref):
    out_ref[...] = lax.dot_general(
        emb_ref[...], w_ref[...],
        (((1,), (1,)), ((), ())),
        preferred_element_type=jnp.float32,
    ) + b_ref[...]


def _project(emb, W, b2):
    return pl.pallas_call(
        _proj_body,
        out_shape=jax.ShapeDtypeStruct((emb.shape[0], W.shape[0]), jnp.float32),
    )(emb, W, b2)


def _make_gather():
    mesh = plsc.VectorSubcoreMesh(core_axis_name="c", subcore_axis_name="s")

    @functools.partial(
        pl.kernel, mesh=mesh,
        out_type=jax.ShapeDtypeStruct((NB, L, D), jnp.float32),
        scratch_types=(
            [pltpu.VMEM((BPW, L), jnp.int32)]
            + [pltpu.VMEM((L, D), jnp.float32)] * NBUF
            + [pltpu.SemaphoreType.DMA] * (2 * NBUF)
        ),
    )
    def gather(table_hbm, idx_hbm, out_hbm, idx_v, *bufs_and_sems):
        rows = bufs_and_sems[:NBUF]
        gsem = bufs_and_sems[NBUF:2 * NBUF]
        osem = bufs_and_sems[2 * NBUF:]
        wid = lax.axis_index("s") * NC + lax.axis_index("c")
        base = wid * BPW
        pltpu.sync_copy(idx_hbm.at[pl.ds(base, BPW)], idx_v)

        def gdesc(c, b):
            return pltpu.make_async_copy(
                table_hbm.at[idx_v.at[c]], rows[b], gsem[b])

        def odesc(c, b):
            return pltpu.make_async_copy(
                rows[b], out_hbm.at[base + c], osem[b])

        for c0 in range(DEPTH):
            gdesc(c0, c0).start()

        def body(g, carry):
            for b in range(NBUF):
                c = NBUF * g + b
                gdesc(c, b).wait()
                odesc(c, b).start()

                bn = (b + DEPTH) % NBUF

                @pl.when(c + DEPTH < BPW)
                def _():
                    @pl.when(c >= 1)
                    def _():
                        odesc(c - 1, bn).wait()

                    gdesc(c + DEPTH, bn).start()
            return carry

        lax.fori_loop(0, BPW // NBUF, body, 0)
        for b in range(NBUF):
            odesc(0, b).wait()

    return gather


_gather = _make_gather()


def kernel(x, emb_table, W, b):
    P = _project(emb_table, W, b.reshape(1, D))
    return _gather(P, x.astype(jnp.int32))


# speedup vs baseline: 2.8804x; 2.3666x over previous
import jax
import jax.numpy as jnp
from jax import lax
from jax.experimental import pallas as pl

NB = 4096
L = 20
D = 512
NV = 100
BN = 128


def _proj_body(emb_ref, w_ref, b_ref, out_ref):
    out_ref[...] = lax.dot_general(
        emb_ref[...], w_ref[...],
        (((1,), (1,)), ((), ())),
        preferred_element_type=jnp.float32,
    ) + b_ref[...]


def _project(emb, W, b2):
    return pl.pallas_call(
        _proj_body,
        out_shape=jax.ShapeDtypeStruct((emb.shape[0], W.shape[0]), jnp.float32),
    )(emb, W, b2)


def _lookup_body(idx_ref, p_ref, out_ref):
    idx = idx_ref[...]
    hot = (idx[:, :, None]
           == lax.broadcasted_iota(jnp.int32, (BN, L, NV), 2)
           ).astype(jnp.float32)
    out_ref[...] = lax.dot_general(
        hot, p_ref[...],
        (((2,), (0,)), ((), ())),
        preferred_element_type=jnp.float32,
    )


def _lookup(idx, P):
    return pl.pallas_call(
        _lookup_body,
        grid=(NB // BN,),
        in_specs=[
            pl.BlockSpec((BN, L), lambda i: (i, 0)),
            pl.BlockSpec((NV, D), lambda i: (0, 0)),
        ],
        out_specs=pl.BlockSpec((BN, L, D), lambda i: (i, 0, 0)),
        out_shape=jax.ShapeDtypeStruct((NB, L, D), jnp.float32),
    )(idx, P)


def kernel(x, emb_table, W, b):
    P = _project(emb_table, W, b.reshape(1, D))
    return _lookup(x.astype(jnp.int32), P)
